# baseline (speedup 1.0000x reference)
import dataclasses
import functools

import jax
import jax.numpy as jnp
from jax import lax
from jax.experimental import pallas as pl
from jax.experimental.pallas import tpu as pltpu
from jax.experimental.pallas import tpu_sc as plsc

IGNORE_INDEX = 255
L_MAIN, L_AUX, L_AFF, L_DIST, L_BDY = 1.0, 1.0, 0.1, 0.1, 0.5


def _sc_pair_d2(rf_flat, jf_flat, idx_flat):
    BN, C = rf_flat.shape
    P = idx_flat.shape[1]
    W = 128
    K = P // BN
    TW = W // K
    L = 16

    mesh = plsc.VectorSubcoreMesh(core_axis_name="core", subcore_axis_name="subcore")
    cp = pltpu.CompilerParams()
    if "needs_layout_passes" in pltpu.CompilerParams.__dataclass_fields__:
        cp = dataclasses.replace(cp, needs_layout_passes=False)

    @functools.partial(
        pl.kernel,
        out_type=(jax.ShapeDtypeStruct((P,), jnp.float32),
                  jax.ShapeDtypeStruct((P,), jnp.float32)),
        mesh=mesh,
        compiler_params=cp,
        scratch_types=[pltpu.VMEM((W, C), jnp.float32),
                       pltpu.VMEM((W, C), jnp.float32)],
    )
    def sck(rf_hbm, jf_hbm, idx_hbm, drf_hbm, djf_hbm, nbr_rf, nbr_jf):
        def body(idx_v, ctr_rf, ctr_jf, drf_v, djf_v):
            pltpu.sync_copy(rf_hbm.at[idx_v.at[0]], nbr_rf)
            pltpu.sync_copy(jf_hbm.at[idx_v.at[0]], nbr_jf)
            lanes = lax.iota(jnp.int32, L)
            toks_off = lax.shift_right_logical(lanes, 3)
            for g in range(W // L):
                rows = g * L + lanes
                toks = (g * L) // K + toks_off
                zero = jnp.zeros((L,), jnp.float32)

                def cbody(c, accs, rows=rows, toks=toks):
                    arf, ajf = accs
                    cs = jnp.full((L,), 0, jnp.int32) + c
                    nv = plsc.load_gather(nbr_rf, [rows, cs])
                    cv = plsc.load_gather(ctr_rf, [toks, cs])
                    d = cv - nv
                    nv2 = plsc.load_gather(nbr_jf, [rows, cs])
                    cv2 = plsc.load_gather(ctr_jf, [toks, cs])
                    d2 = cv2 - nv2
                    return (arf + d * d, ajf + d2 * d2)

                arf, ajf = lax.fori_loop(0, C, cbody, (zero, zero))
                drf_v[pl.ds(g * L, L)] = arf
                djf_v[pl.ds(g * L, L)] = ajf

        pltpu.emit_pipeline(
            body,
            grid=(P // W,),
            in_specs=[pl.BlockSpec((1, W), lambda i: (0, i)),
                      pl.BlockSpec((TW, C), lambda i: (i, 0)),
                      pl.BlockSpec((TW, C), lambda i: (i, 0))],
            out_specs=[pl.BlockSpec((W,), lambda i: (i,)),
                       pl.BlockSpec((W,), lambda i: (i,))],
            core_axis_name=("core", "subcore"),
            dimension_semantics=(pltpu.PARALLEL,),
        )(idx_hbm, rf_hbm, jf_hbm, drf_hbm, djf_hbm)

    return sck(rf_flat, jf_flat, idx_flat)


def _tc_losses(rlog, alog, tgt2d, feat, prot, aff, bdy, drf, djf):
    BN, NC = rlog.shape
    C = feat.shape[1]
    K = aff.shape[1]
    R = 2048
    G = BN // R

    def body(rlog_r, alog_r, tgt_r, feat_r, prot_r, aff_r, bdy_r, drf_r, djf_r,
             out_r, acc):
        i = pl.program_id(0)

        @pl.when(i == 0)
        def _init():
            for j in range(8):
                acc[j] = 0.0

        tgt = tgt_r[...]
        valid = tgt != IGNORE_INDEX
        tgt0 = jnp.where(valid, tgt, 0)
        iota = lax.broadcasted_iota(jnp.int32, (R, NC), 1)
        onehot = iota == tgt0
        validf = valid.astype(jnp.float32)

        def ce_sum(lg):
            m = jnp.max(lg, axis=1, keepdims=True)
            l = lg - m
            lse = jnp.log(jnp.sum(jnp.exp(l), axis=1, keepdims=True))
            ltgt = jnp.sum(jnp.where(onehot, l, 0.0), axis=1, keepdims=True)
            return jnp.sum(jnp.where(valid, lse - ltgt, 0.0))

        s_main = ce_sum(rlog_r[...])
        s_aux = ce_sum(alog_r[...])
        n_valid = jnp.sum(validf)

        f = feat_r[...]
        p = prot_r[...]
        pn = p / jnp.maximum(jnp.sqrt(jnp.sum(p * p, axis=1, keepdims=True)), 1e-12)
        sim = lax.dot_general(f, pn, (((1,), (1,)), ((), ())),
                              preferred_element_type=jnp.float32)
        fnorm = jnp.maximum(jnp.sqrt(jnp.sum(f * f, axis=1, keepdims=True)), 1e-12)
        tsim = jnp.sum(jnp.where(onehot, sim, 0.0), axis=1, keepdims=True) / fnorm
        s_dist = jnp.sum(jnp.where(valid, 1.0 - tsim, 0.0))

        w = jnp.maximum(aff_r[...] - 0.5, 0.0)
        s_affn = jnp.sum(w * drf_r[...])
        s_affd = jnp.sum(w)

        es = jnp.mean(jnp.sqrt(djf_r[...]), axis=1, keepdims=True)
        tb = jax.nn.sigmoid((es - 0.15) * 20.0)
        x = bdy_r[...]
        bce = jnp.maximum(x, 0.0) - x * tb + jnp.log1p(jnp.exp(-jnp.abs(x)))
        s_bdy = jnp.sum(bce)

        acc[0] += s_main
        acc[1] += s_aux
        acc[2] += s_dist
        acc[3] += s_affn
        acc[4] += s_affd
        acc[5] += s_bdy
        acc[6] += n_valid

        denom = jnp.maximum(acc[6], 1.0)
        loss = L_MAIN * acc[0] / denom + L_AUX * acc[1] / denom
        loss += L_AFF * (acc[3] / (C ** 0.5)) / (acc[4] + 0.0001)
        loss += L_DIST * acc[2] / denom
        loss += L_BDY * acc[5] / BN
        out_r[0, 0] = loss

    return pl.pallas_call(
        body,
        grid=(G,),
        in_specs=[
            pl.BlockSpec((R, NC), lambda i: (i, 0)),
            pl.BlockSpec((R, NC), lambda i: (i, 0)),
            pl.BlockSpec((R, 1), lambda i: (i, 0)),
            pl.BlockSpec((R, C), lambda i: (i, 0)),
            pl.BlockSpec((NC, C), lambda i: (0, 0)),
            pl.BlockSpec((R, K), lambda i: (i, 0)),
            pl.BlockSpec((R, 1), lambda i: (i, 0)),
            pl.BlockSpec((R, K), lambda i: (i, 0)),
            pl.BlockSpec((R, K), lambda i: (i, 0)),
        ],
        out_specs=pl.BlockSpec((1, 1), lambda i: (0, 0),
                               memory_space=pltpu.SMEM),
        out_shape=jax.ShapeDtypeStruct((1, 1), jnp.float32),
        scratch_shapes=[pltpu.SMEM((8,), jnp.float32)],
    )(rlog, alog, tgt2d, feat, prot, aff, bdy, drf, djf)


def kernel(refined_logits, aux_logits, refined_feat, affinity, prototypes,
           input_jafar_feat, bdy_logits, target, k_idx):
    B, N, K = k_idx.shape
    C = refined_feat.shape[-1]
    BN = B * N

    batch_offset = (jnp.arange(B, dtype=jnp.int32) * N).reshape(B, 1, 1)
    idx_flat = (k_idx + batch_offset).reshape(1, BN * K)
    rf_flat = refined_feat.reshape(BN, C)
    jf_flat = input_jafar_feat.reshape(BN, C)

    d_rf, d_jf = _sc_pair_d2(rf_flat, jf_flat, idx_flat)

    out = _tc_losses(refined_logits, aux_logits, target.reshape(BN, 1),
                     rf_flat, prototypes, affinity.reshape(BN, K),
                     bdy_logits.reshape(BN, 1), d_rf.reshape(BN, K),
                     d_jf.reshape(BN, K))
    return out[0, 0]


# speedup vs baseline: 10.0070x; 10.0070x over previous
import dataclasses
import functools

import jax
import jax.numpy as jnp
from jax import lax
from jax.experimental import pallas as pl
from jax.experimental.pallas import tpu as pltpu
from jax.experimental.pallas import tpu_sc as plsc

IGNORE_INDEX = 255
L_MAIN, L_AUX, L_AFF, L_DIST, L_BDY = 1.0, 1.0, 0.1, 0.1, 0.5


def _sc_pair_d2(rf_flat, jf_flat, idx_flat):
    BN, C = rf_flat.shape
    P = idx_flat.shape[1]
    K = P // BN
    L = 16
    NSUB = 32
    PPS = P // NSUB
    TPS = BN // NSUB
    Wp = 64
    NW = PPS // Wp
    TW = Wp // K
    CH = C // L

    mesh = plsc.VectorSubcoreMesh(core_axis_name="core", subcore_axis_name="subcore")
    cp = pltpu.CompilerParams()
    if "needs_layout_passes" in pltpu.CompilerParams.__dataclass_fields__:
        cp = dataclasses.replace(cp, needs_layout_passes=False)

    @functools.partial(
        pl.kernel,
        out_type=(jax.ShapeDtypeStruct((P,), jnp.float32),
                  jax.ShapeDtypeStruct((P,), jnp.float32)),
        mesh=mesh,
        compiler_params=cp,
        scratch_types=[
            pltpu.VMEM((PPS,), jnp.int32),
            pltpu.VMEM((2, Wp, C), jnp.float32),
            pltpu.VMEM((2, Wp, C), jnp.float32),
            pltpu.VMEM((2, TW, C), jnp.float32),
            pltpu.VMEM((2, TW, C), jnp.float32),
            pltpu.VMEM((PPS,), jnp.float32),
            pltpu.VMEM((PPS,), jnp.float32),
            pltpu.SemaphoreType.DMA((2,)),
            pltpu.SemaphoreType.DMA((2,)),
            pltpu.SemaphoreType.DMA((2,)),
            pltpu.SemaphoreType.DMA((2,)),
        ],
    )
    def sck(rf_hbm, jf_hbm, idx_hbm, drf_hbm, djf_hbm,
            idx_all, nbr_rf, nbr_jf, ctr_rf, ctr_jf, drf_all, djf_all,
            s_grf, s_gjf, s_crf, s_cjf):
        wid = lax.axis_index("subcore") * 2 + lax.axis_index("core")
        pbase = wid * PPS
        tbase = wid * TPS
        pltpu.sync_copy(idx_hbm.at[0, pl.ds(pbase, PPS)], idx_all)

        def dmas(w, b):
            iv = idx_all.at[pl.ds(w * Wp, Wp)]
            t0 = tbase + w * TW
            return (
                pltpu.make_async_copy(rf_hbm.at[iv], nbr_rf.at[b], s_grf.at[b]),
                pltpu.make_async_copy(jf_hbm.at[iv], nbr_jf.at[b], s_gjf.at[b]),
                pltpu.make_async_copy(rf_hbm.at[pl.ds(t0, TW)], ctr_rf.at[b], s_crf.at[b]),
                pltpu.make_async_copy(jf_hbm.at[pl.ds(t0, TW)], ctr_jf.at[b], s_cjf.at[b]),
            )

        def issue(w, b):
            for d in dmas(w, b):
                d.start()

        def wait(w, b):
            for d in dmas(w, b):
                d.wait()

        def compute(w, b):
            obase = w * Wp
            lanes = lax.iota(jnp.int32, L)

            def gbody(g, carry):
                vrf = jnp.zeros((L,), jnp.float32)
                vjf = jnp.zeros((L,), jnp.float32)
                for j in range(L):
                    pp = g * L + j
                    t = g * (L // K) + (j // K)
                    arf = jnp.zeros((L,), jnp.float32)
                    ajf = jnp.zeros((L,), jnp.float32)
                    for cc in range(CH):
                        s = pl.ds(cc * L, L)
                        d = ctr_rf[b, t, s] - nbr_rf[b, pp, s]
                        arf = arf + d * d
                        d2 = ctr_jf[b, t, s] - nbr_jf[b, pp, s]
                        ajf = ajf + d2 * d2
                    vrf = jnp.where(lanes == j, jnp.sum(arf), vrf)
                    vjf = jnp.where(lanes == j, jnp.sum(ajf), vjf)
                drf_all[pl.ds(obase + g * L, L)] = vrf
                djf_all[pl.ds(obase + g * L, L)] = vjf
                return carry

            lax.fori_loop(0, Wp // L, gbody, 0)

        issue(0, 0)

        @pl.loop(0, NW, step=2)
        def _(w):
            for b in (0, 1):
                we = w + b

                @pl.when(we + 1 < NW)
                def _issue_next(we=we, b=b):
                    issue(we + 1, b ^ 1)

                wait(we, b)
                compute(we, b)

        pltpu.sync_copy(drf_all, drf_hbm.at[pl.ds(pbase, PPS)])
        pltpu.sync_copy(djf_all, djf_hbm.at[pl.ds(pbase, PPS)])

    return sck(rf_flat, jf_flat, idx_flat)


def _tc_losses(rlog, alog, tgt2d, feat, prot, aff, bdy, drf, djf):
    BN, NC = rlog.shape
    C = feat.shape[1]
    K = aff.shape[1]
    R = 2048
    G = BN // R

    def body(rlog_r, alog_r, tgt_r, feat_r, prot_r, aff_r, bdy_r, drf_r, djf_r,
             out_r, acc):
        i = pl.program_id(0)

        @pl.when(i == 0)
        def _init():
            for j in range(8):
                acc[j] = 0.0

        tgt = tgt_r[...]
        valid = tgt != IGNORE_INDEX
        tgt0 = jnp.where(valid, tgt, 0)
        iota = lax.broadcasted_iota(jnp.int32, (R, NC), 1)
        onehot = iota == tgt0
        validf = valid.astype(jnp.float32)

        def ce_sum(lg):
            m = jnp.max(lg, axis=1, keepdims=True)
            l = lg - m
            lse = jnp.log(jnp.sum(jnp.exp(l), axis=1, keepdims=True))
            ltgt = jnp.sum(jnp.where(onehot, l, 0.0), axis=1, keepdims=True)
            return jnp.sum(jnp.where(valid, lse - ltgt, 0.0))

        s_main = ce_sum(rlog_r[...])
        s_aux = ce_sum(alog_r[...])
        n_valid = jnp.sum(validf)

        f = feat_r[...]
        p = prot_r[...]
        pn = p / jnp.maximum(jnp.sqrt(jnp.sum(p * p, axis=1, keepdims=True)), 1e-12)
        sim = lax.dot_general(f, pn, (((1,), (1,)), ((), ())),
                              preferred_element_type=jnp.float32)
        fnorm = jnp.maximum(jnp.sqrt(jnp.sum(f * f, axis=1, keepdims=True)), 1e-12)
        tsim = jnp.sum(jnp.where(onehot, sim, 0.0), axis=1, keepdims=True) / fnorm
        s_dist = jnp.sum(jnp.where(valid, 1.0 - tsim, 0.0))

        w = jnp.maximum(aff_r[...] - 0.5, 0.0)
        s_affn = jnp.sum(w * drf_r[...])
        s_affd = jnp.sum(w)

        es = jnp.mean(jnp.sqrt(djf_r[...]), axis=1, keepdims=True)
        tb = jax.nn.sigmoid((es - 0.15) * 20.0)
        x = bdy_r[...]
        bce = jnp.maximum(x, 0.0) - x * tb + jnp.log1p(jnp.exp(-jnp.abs(x)))
        s_bdy = jnp.sum(bce)

        acc[0] += s_main
        acc[1] += s_aux
        acc[2] += s_dist
        acc[3] += s_affn
        acc[4] += s_affd
        acc[5] += s_bdy
        acc[6] += n_valid

        denom = jnp.maximum(acc[6], 1.0)
        loss = L_MAIN * acc[0] / denom + L_AUX * acc[1] / denom
        loss += L_AFF * (acc[3] / (C ** 0.5)) / (acc[4] + 0.0001)
        loss += L_DIST * acc[2] / denom
        loss += L_BDY * acc[5] / BN
        out_r[0, 0] = loss

    return pl.pallas_call(
        body,
        grid=(G,),
        in_specs=[
            pl.BlockSpec((R, NC), lambda i: (i, 0)),
            pl.BlockSpec((R, NC), lambda i: (i, 0)),
            pl.BlockSpec((R, 1), lambda i: (i, 0)),
            pl.BlockSpec((R, C), lambda i: (i, 0)),
            pl.BlockSpec((NC, C), lambda i: (0, 0)),
            pl.BlockSpec((R, K), lambda i: (i, 0)),
            pl.BlockSpec((R, 1), lambda i: (i, 0)),
            pl.BlockSpec((R, K), lambda i: (i, 0)),
            pl.BlockSpec((R, K), lambda i: (i, 0)),
        ],
        out_specs=pl.BlockSpec((1, 1), lambda i: (0, 0),
                               memory_space=pltpu.SMEM),
        out_shape=jax.ShapeDtypeStruct((1, 1), jnp.float32),
        scratch_shapes=[pltpu.SMEM((8,), jnp.float32)],
    )(rlog, alog, tgt2d, feat, prot, aff, bdy, drf, djf)


def kernel(refined_logits, aux_logits, refined_feat, affinity, prototypes,
           input_jafar_feat, bdy_logits, target, k_idx):
    B, N, K = k_idx.shape
    C = refined_feat.shape[-1]
    BN = B * N

    batch_offset = (jnp.arange(B, dtype=jnp.int32) * N).reshape(B, 1, 1)
    idx_flat = (k_idx + batch_offset).reshape(1, BN * K)
    rf_flat = refined_feat.reshape(BN, C)
    jf_flat = input_jafar_feat.reshape(BN, C)

    d_rf, d_jf = _sc_pair_d2(rf_flat, jf_flat, idx_flat)

    out = _tc_losses(refined_logits, aux_logits, target.reshape(BN, 1),
                     rf_flat, prototypes, affinity.reshape(BN, K),
                     bdy_logits.reshape(BN, 1), d_rf.reshape(BN, K),
                     d_jf.reshape(BN, K))
    return out[0, 0]
